# Initial kernel scaffold; baseline (speedup 1.0000x reference)
import jax
import jax.numpy as jnp
from jax.experimental import pallas as pl


def kernel(ids, table):
    raise NotImplementedError("write your pallas kernel here")


# speedup vs baseline: 2.1526x; 2.1526x over previous
import functools

import jax
import jax.numpy as jnp
from jax import lax
from jax.experimental import pallas as pl
from jax.experimental.pallas import tpu as pltpu
from jax.experimental.pallas import tpu_sc as plsc

_VOCAB = 1000000
_D = 32
_B = 16384
_F = 26
_N = _B * _F
_NW = 32
_PER_W = _N // _NW
_C = 512
_NCH = _PER_W // _C
_G = 128
_NG = _C // _G

_mesh = plsc.VectorSubcoreMesh(core_axis_name="c", subcore_axis_name="s")


@functools.partial(
    pl.kernel,
    out_type=jax.ShapeDtypeStruct((_N, _D), jnp.float32),
    mesh=_mesh,
    compiler_params=pltpu.CompilerParams(use_tc_tiling_on_sc=False),
    scratch_types=[
        pltpu.VMEM((_C,), jnp.int32),
        pltpu.VMEM((_C,), jnp.int32),
        pltpu.VMEM((_C,), jnp.int32),
        pltpu.VMEM((_C, _D), jnp.float32),
        pltpu.VMEM((_C, _D), jnp.float32),
        pltpu.SemaphoreType.DMA,
    ],
)
def _emb_kernel(ids_hbm, table_hbm, out_hbm, ids_v, idx0_v, idx1_v, buf0, buf1, sem):
    cid = lax.axis_index("c")
    sid = lax.axis_index("s")
    wid = sid * 2 + cid
    base = wid * _PER_W

    def chunk_body(g, carry):
        off = base + g * _C
        pltpu.sync_copy(ids_hbm.at[pl.ds(off, _C)], ids_v)

        def idx_body(i, c2):
            sl = pl.ds(i * 16, 16)
            v = ids_v[sl]
            idx1_v[sl] = lax.bitwise_and(v, jnp.int32(65535))
            m0 = lax.bitwise_and(v, jnp.int32(-65536))
            q = (m0.astype(jnp.float32) * jnp.float32(1.0 / _VOCAB)).astype(jnp.int32)
            r = m0 - q * jnp.int32(_VOCAB)
            r = jnp.where(r < 0, r + jnp.int32(_VOCAB), r)
            r = jnp.where(r >= jnp.int32(_VOCAB), r - jnp.int32(_VOCAB), r)
            idx0_v[sl] = r
            return c2

        lax.fori_loop(0, _C // 16, idx_body, 0)

        copies = []
        for j in range(_NG):
            jsl = pl.ds(j * _G, _G)
            copies.append(pltpu.async_copy(table_hbm.at[idx0_v.at[jsl]], buf0.at[jsl], sem))
            copies.append(pltpu.async_copy(table_hbm.at[idx1_v.at[jsl]], buf1.at[jsl], sem))
        for cp in copies:
            cp.wait()

        def add_body(r_, c2):
            for h in range(_D // 16):
                sl = pl.ds(h * 16, 16)
                buf0[r_, sl] = buf0[r_, sl] + buf1[r_, sl]
            return c2

        lax.fori_loop(0, _C, add_body, 0)

        pltpu.sync_copy(buf0, out_hbm.at[pl.ds(off, _C)])
        return carry

    lax.fori_loop(0, _NCH, chunk_body, 0)


def kernel(ids, table):
    out = _emb_kernel(ids.reshape(_N), table)
    return out.reshape(_B, _F, _D)


# speedup vs baseline: 2.3897x; 1.1102x over previous
import functools

import jax
import jax.numpy as jnp
from jax import lax
from jax.experimental import pallas as pl
from jax.experimental.pallas import tpu as pltpu
from jax.experimental.pallas import tpu_sc as plsc

_VOCAB = 1000000
_D = 32
_B = 16384
_F = 26
_N = _B * _F
_NW = 32
_PER_W = _N // _NW
_C = 512
_NCH = _PER_W // _C
_G = 128
_NG = _C // _G
_NP = 3

_mesh = plsc.VectorSubcoreMesh(core_axis_name="c", subcore_axis_name="s")


@functools.partial(
    pl.kernel,
    out_type=jax.ShapeDtypeStruct((_N, _D), jnp.float32),
    mesh=_mesh,
    compiler_params=pltpu.CompilerParams(use_tc_tiling_on_sc=False),
    scratch_types=[
        pltpu.VMEM((_PER_W,), jnp.int32),
        pltpu.VMEM((_NP, _C), jnp.int32),
        pltpu.VMEM((_NP, _C), jnp.int32),
        pltpu.VMEM((_NP, _C, _D), jnp.float32),
        pltpu.VMEM((_NP, _C, _D), jnp.float32),
        pltpu.SemaphoreType.DMA,
        pltpu.SemaphoreType.DMA,
        pltpu.SemaphoreType.DMA,
        pltpu.SemaphoreType.DMA,
        pltpu.SemaphoreType.DMA,
        pltpu.SemaphoreType.DMA,
    ],
)
def _emb_kernel(ids_hbm, table_hbm, out_hbm, ids_v, idx0_v, idx1_v, buf0, buf1,
                gsem0, gsem1, gsem2, osem0, osem1, osem2):
    cid = lax.axis_index("c")
    sid = lax.axis_index("s")
    base = (sid * 2 + cid) * _PER_W
    gsems = (gsem0, gsem1, gsem2)
    osems = (osem0, osem1, osem2)

    pltpu.sync_copy(ids_hbm.at[pl.ds(base, _PER_W)], ids_v)

    def stage_a(g, p, drain_pred):
        def drain_out():
            pltpu.make_async_copy(out_hbm.at[pl.ds(0, _C)], buf1.at[p],
                                  osems[p]).wait()

        if drain_pred is None:
            drain_out()
        elif drain_pred is not False:
            pl.when(drain_pred)(drain_out)

        goff = g * _C

        @plsc.parallel_loop(0, _C // 16, unroll=4)
        def _idx_body(i):
            v = ids_v[pl.ds(goff + i * 16, 16)]
            idx1_v[p, pl.ds(i * 16, 16)] = lax.bitwise_and(v, jnp.int32(65535))
            m0 = lax.bitwise_and(v, jnp.int32(-65536))
            q = (m0.astype(jnp.float32) * jnp.float32(1.0 / _VOCAB)).astype(jnp.int32)
            r = m0 - q * jnp.int32(_VOCAB)
            r = jnp.where(r < 0, r + jnp.int32(_VOCAB), r)
            r = jnp.where(r >= jnp.int32(_VOCAB), r - jnp.int32(_VOCAB), r)
            idx0_v[p, pl.ds(i * 16, 16)] = r

        for j in range(_NG):
            jsl = pl.ds(j * _G, _G)
            pltpu.async_copy(table_hbm.at[idx0_v.at[p, jsl]],
                             buf0.at[p, jsl], gsems[p])
            pltpu.async_copy(table_hbm.at[idx1_v.at[p, jsl]],
                             buf1.at[p, jsl], gsems[p])

    def stage_b(g, p):
        pltpu.make_async_copy(table_hbm.at[pl.ds(0, _C)], buf0.at[p],
                              gsems[p]).wait()
        pltpu.make_async_copy(table_hbm.at[pl.ds(0, _C)], buf1.at[p],
                              gsems[p]).wait()

        @plsc.parallel_loop(0, _C, unroll=8)
        def _add_body(r_):
            for h in range(_D // 16):
                sl = pl.ds(h * 16, 16)
                plsc.addupdate(buf0.at[p, r_, sl], buf1[p, r_, sl])

        pltpu.async_copy(buf0.at[p], out_hbm.at[pl.ds(base + g * _C, _C)],
                         osems[p])

    stage_a(0, 0, False)
    stage_a(1, 1, False)

    def outer(go, carry):
        g0 = 3 * go
        for k in range(3):
            g = g0 + k
            stage_b(g, k)
            pa = (k + 2) % 3
            stage_a(g + 2, pa, (go > 0) if k == 0 else None)
        return carry

    lax.fori_loop(0, (_NCH - 2) // 3, outer, 0)

    stage_b(_NCH - 2, 0)
    stage_b(_NCH - 1, 1)
    for p in (2, 0, 1):
        pltpu.make_async_copy(out_hbm.at[pl.ds(0, _C)], buf1.at[p],
                              osems[p]).wait()


def kernel(ids, table):
    out = _emb_kernel(ids.reshape(_N), table)
    return out.reshape(_B, _F, _D)


# speedup vs baseline: 4.7504x; 1.9879x over previous
import functools

import jax
import jax.numpy as jnp
from jax import lax
from jax.experimental import pallas as pl
from jax.experimental.pallas import tpu as pltpu
from jax.experimental.pallas import tpu_sc as plsc

_D = 32
_B = 16384
_F = 26
_NW = 32
_C0 = 15625
_C1 = 65536
_CC = 81280
_VB = _CC // 128
_BT = _B // 128
_UNITS = _F * _BT
_UPW = _UNITS // _NW
_BPW = 4

_mesh = plsc.VectorSubcoreMesh(core_axis_name="c", subcore_axis_name="s")


@functools.partial(
    pl.kernel,
    out_type=jax.ShapeDtypeStruct((_CC * _D // 128, 128), jnp.float32),
    mesh=_mesh,
    compiler_params=pltpu.CompilerParams(use_tc_tiling_on_sc=True, needs_layout_passes=False),
    scratch_types=[
        pltpu.VMEM((2, 4, 8, 128), jnp.float32),
        pltpu.VMEM((2, 32, 128), jnp.float32),
        pltpu.SemaphoreType.DMA,
        pltpu.SemaphoreType.DMA,
        pltpu.SemaphoreType.DMA,
        pltpu.SemaphoreType.DMA,
    ],
)
def _relayout(src_hbm, lin_hbm, vin, vout, gsem0, gsem1, osem0, osem1):
    cid = lax.axis_index("c")
    sid = lax.axis_index("s")
    wid = sid * 2 + cid
    gsems = (gsem0, gsem1)
    osems = (osem0, osem1)
    nfull = 19

    def fire(n):
        p = n % 2
        vb = wid + 32 * n
        for dt in range(4):
            pltpu.async_copy(
                src_hbm.at[pl.ds(dt * 8, 8), pl.ds(vb * 128, 128)],
                vin.at[p, dt], gsems[p])

    def retire(n):
        p = n % 2
        vb = wid + 32 * n
        for dt in range(4):
            pltpu.make_async_copy(lin_hbm.at[pl.ds(0, 8)], vin.at[p, dt],
                                  gsems[p]).wait()
        if n >= 2:
            pltpu.make_async_copy(lin_hbm.at[pl.ds(0, 32)], vout.at[p],
                                  osems[p]).wait()

        @plsc.parallel_loop(0, 256, unroll=4)
        def _asm(j):
            mr = lax.shift_right_logical(j, 3)
            jj = lax.bitwise_and(j, 7)
            lane0 = jj * 16
            d = lax.bitwise_and(lane0, 31) + lax.iota(jnp.int32, 16)
            dt_v = lax.shift_right_logical(d, 3)
            di_v = lax.bitwise_and(d, 7)
            vi = jnp.full((16,), 0, jnp.int32) + (mr * 4 + lax.shift_right_logical(jj, 1))
            vout[p, mr, pl.ds(lane0, 16)] = plsc.load_gather(
                vin.at[p], [dt_v, di_v, vi])

        pltpu.async_copy(vout.at[p], lin_hbm.at[pl.ds(vb * 32, 32)], osems[p])

    fire(0)
    for n in range(1, nfull):
        fire(n)
        retire(n - 1)

    @pl.when(wid < 27)
    def _():
        fire(19)
    retire(nfull - 1)

    @pl.when(wid < 27)
    def _():
        retire(19)

    pltpu.make_async_copy(lin_hbm.at[pl.ds(0, 32)], vout.at[0],
                          osems[0]).wait()
    pltpu.make_async_copy(lin_hbm.at[pl.ds(0, 32)], vout.at[1],
                          osems[1]).wait()


@functools.partial(
    pl.kernel,
    out_type=jax.ShapeDtypeStruct((_F, 4, 128, 8, 128), jnp.float32),
    mesh=_mesh,
    compiler_params=pltpu.CompilerParams(use_tc_tiling_on_sc=False, needs_layout_passes=False),
    scratch_types=[
        pltpu.VMEM((_F, 512), jnp.int32),
        pltpu.VMEM((3, 128), jnp.int32),
        pltpu.VMEM((3, 128), jnp.int32),
        pltpu.VMEM((3, 128, _D), jnp.float32),
        pltpu.VMEM((3, 128, _D), jnp.float32),
        pltpu.VMEM((3, 4, 8, 128), jnp.float32),
        pltpu.SemaphoreType.DMA,
        pltpu.SemaphoreType.DMA,
        pltpu.SemaphoreType.DMA,
        pltpu.SemaphoreType.DMA,
        pltpu.SemaphoreType.DMA,
        pltpu.SemaphoreType.DMA,
    ],
)
def _gather(ids_hbm, ctab_hbm, out_hbm, ids_v, idx0_v, idx1_v, buf0, buf1,
            vout, gsem0, gsem1, gsem2, osem0, osem1, osem2):
    cid = lax.axis_index("c")
    sid = lax.axis_index("s")
    wid = sid * 2 + cid
    gsems = (gsem0, gsem1, gsem2)
    osems = (osem0, osem1, osem2)

    pltpu.sync_copy(ids_hbm.at[:, pl.ds(wid * 512, 512)], ids_v)

    def stage_a(f, btl, p):
        for i in range(8):
            sl = pl.ds(i * 16, 16)
            v = ids_v[f, pl.ds(btl * 128 + i * 16, 16)]
            idx1_v[p, sl] = lax.bitwise_and(v, jnp.int32(65535)) + jnp.int32(_C0)
            h = lax.shift_right_logical(v, 16)
            x = h * jnp.int32(1024) + jnp.where(
                h >= jnp.int32(32768), jnp.int32(511), jnp.int32(0))
            q = (x.astype(jnp.float32) * jnp.float32(1.0 / 15625)).astype(jnp.int32)
            r = x - q * jnp.int32(15625)
            r = jnp.where(r < 0, r + jnp.int32(15625), r)
            r = jnp.where(r >= jnp.int32(15625), r - jnp.int32(15625), r)
            idx0_v[p, sl] = r
        pltpu.async_copy(ctab_hbm.at[idx0_v.at[p]], buf0.at[p], gsems[p])
        pltpu.async_copy(ctab_hbm.at[idx1_v.at[p]], buf1.at[p], gsems[p])

    def stage_b(f, bt, p, first):
        pltpu.make_async_copy(ctab_hbm.at[pl.ds(0, 128)], buf0.at[p],
                              gsems[p]).wait()
        pltpu.make_async_copy(ctab_hbm.at[pl.ds(0, 128)], buf1.at[p],
                              gsems[p]).wait()

        def drain_out():
            for dt in range(4):
                pltpu.make_async_copy(out_hbm.at[0, 0, 0], vout.at[p, dt],
                                      osems[p]).wait()

        if first is None:
            drain_out()
        else:
            pl.when(first)(drain_out)

        @plsc.parallel_loop(0, 128, unroll=4)
        def _asm(r_):
            bi = jnp.full((16,), 0, jnp.int32) + r_
            for half in range(2):
                sl = pl.ds(half * 16, 16)
                s = buf0[p, r_, sl] + buf1[p, r_, sl]
                d = half * 16 + lax.iota(jnp.int32, 16)
                plsc.store_scatter(
                    vout.at[p],
                    [lax.shift_right_logical(d, 3), lax.bitwise_and(d, 7), bi],
                    s)

        for dt in range(4):
            pltpu.async_copy(vout.at[p, dt], out_hbm.at[f, dt, bt], osems[p])

    def advance(f, btl):
        wrap = f >= jnp.int32(_F - 1)
        f2 = jnp.where(wrap, 0, f + 1)
        btl2 = jnp.where(wrap, btl + 1, btl)
        return f2, btl2

    bt0 = wid * _BPW

    stage_a(jnp.int32(0), jnp.int32(0), 0)
    stage_a(jnp.int32(1), jnp.int32(0), 1)

    def outer(go, carry):
        f0, b0 = carry
        f1, b1 = advance(f0, b0)
        f2, b2 = advance(f1, b1)
        f3, b3 = advance(f2, b2)
        f4, b4 = advance(f3, b3)
        fs = (f0, f1, f2, f3, f4)
        bs = (b0, b1, b2, b3, b4)
        for k in range(3):
            stage_b(fs[k], bt0 + bs[k], k, go > 0)
            stage_a(fs[k + 2], bs[k + 2], (k + 2) % 3)
        return f3, b3

    fL, bL = lax.fori_loop(0, (_UPW - 2) // 3, outer,
                           (jnp.int32(0), jnp.int32(0)))

    stage_b(jnp.int32(24), bt0 + jnp.int32(3), 0, None)
    stage_b(jnp.int32(25), bt0 + jnp.int32(3), 1, None)
    for p in (2, 0, 1):
        for dt in range(4):
            pltpu.make_async_copy(out_hbm.at[0, 0, 0], vout.at[p, dt],
                                  osems[p]).wait()


def kernel(ids, table):
    c0 = table[::64]
    c1 = table[:_C1]
    pad = jnp.zeros((_CC - _C0 - _C1, _D), jnp.float32)
    cc = jnp.concatenate([c0, c1, pad], axis=0)
    lin = _relayout(cc.T)
    out5 = _gather(ids.T, lin.reshape(_CC, _D))
    return out5.transpose(2, 4, 0, 1, 3).reshape(_B, _F, _D)


# speedup vs baseline: 4.8437x; 1.0196x over previous
import functools

import jax
import jax.numpy as jnp
from jax import lax
from jax.experimental import pallas as pl
from jax.experimental.pallas import tpu as pltpu
from jax.experimental.pallas import tpu_sc as plsc

_D = 32
_B = 16384
_F = 26
_NW = 32
_C0 = 15625
_C1 = 65536
_CC = 81280
_VB = _CC // 128
_BT = _B // 128
_UNITS = _F * _BT
_UPW = _UNITS // _NW
_BPW = 4

_mesh = plsc.VectorSubcoreMesh(core_axis_name="c", subcore_axis_name="s")


@functools.partial(
    pl.kernel,
    out_type=jax.ShapeDtypeStruct((_CC * _D // 128, 128), jnp.float32),
    mesh=_mesh,
    compiler_params=pltpu.CompilerParams(use_tc_tiling_on_sc=True, needs_layout_passes=False),
    scratch_types=[
        pltpu.VMEM((2, 4, 8, 128), jnp.float32),
        pltpu.VMEM((2, 32, 128), jnp.float32),
        pltpu.SemaphoreType.DMA,
        pltpu.SemaphoreType.DMA,
        pltpu.SemaphoreType.DMA,
        pltpu.SemaphoreType.DMA,
    ],
)
def _relayout(src_hbm, lin_hbm, vin, vout, gsem0, gsem1, osem0, osem1):
    cid = lax.axis_index("c")
    sid = lax.axis_index("s")
    wid = sid * 2 + cid
    gsems = (gsem0, gsem1)
    osems = (osem0, osem1)
    nfull = 19

    def fire(n):
        p = n % 2
        vb = wid + 32 * n
        for dt in range(4):
            pltpu.async_copy(
                src_hbm.at[pl.ds(dt * 8, 8), pl.ds(vb * 128, 128)],
                vin.at[p, dt], gsems[p])

    def retire(n):
        p = n % 2
        vb = wid + 32 * n
        for dt in range(4):
            pltpu.make_async_copy(lin_hbm.at[pl.ds(0, 8)], vin.at[p, dt],
                                  gsems[p]).wait()
        if n >= 2:
            pltpu.make_async_copy(lin_hbm.at[pl.ds(0, 32)], vout.at[p],
                                  osems[p]).wait()

        @plsc.parallel_loop(0, 256, unroll=4)
        def _asm(j):
            mr = lax.shift_right_logical(j, 3)
            jj = lax.bitwise_and(j, 7)
            lane0 = jj * 16
            d = lax.bitwise_and(lane0, 31) + lax.iota(jnp.int32, 16)
            dt_v = lax.shift_right_logical(d, 3)
            di_v = lax.bitwise_and(d, 7)
            vi = jnp.full((16,), 0, jnp.int32) + (mr * 4 + lax.shift_right_logical(jj, 1))
            vout[p, mr, pl.ds(lane0, 16)] = plsc.load_gather(
                vin.at[p], [dt_v, di_v, vi])

        pltpu.async_copy(vout.at[p], lin_hbm.at[pl.ds(vb * 32, 32)], osems[p])

    fire(0)
    for n in range(1, nfull):
        fire(n)
        retire(n - 1)

    @pl.when(wid < 27)
    def _():
        fire(19)
    retire(nfull - 1)

    @pl.when(wid < 27)
    def _():
        retire(19)

    pltpu.make_async_copy(lin_hbm.at[pl.ds(0, 32)], vout.at[0],
                          osems[0]).wait()
    pltpu.make_async_copy(lin_hbm.at[pl.ds(0, 32)], vout.at[1],
                          osems[1]).wait()


@functools.partial(
    pl.kernel,
    out_type=jax.ShapeDtypeStruct((_F, 4, 128, 8, 128), jnp.float32),
    mesh=_mesh,
    compiler_params=pltpu.CompilerParams(use_tc_tiling_on_sc=False, needs_layout_passes=False),
    scratch_types=[
        pltpu.VMEM((_F, 512), jnp.int32),
        pltpu.VMEM((2, 512), jnp.int32),
        pltpu.VMEM((2, 512), jnp.int32),
        pltpu.VMEM((2, 512, _D), jnp.float32),
        pltpu.VMEM((2, 512, _D), jnp.float32),
        pltpu.VMEM((2, 4, 4, 8, 128), jnp.float32),
        pltpu.SemaphoreType.DMA,
        pltpu.SemaphoreType.DMA,
        pltpu.SemaphoreType.DMA,
        pltpu.SemaphoreType.DMA,
    ],
)
def _gather(ids_hbm, ctab_hbm, out_hbm, ids_v, idx0_v, idx1_v, buf0, buf1,
            vout, gsem0, gsem1, osem0, osem1):
    cid = lax.axis_index("c")
    sid = lax.axis_index("s")
    wid = sid * 2 + cid
    gsems = (gsem0, gsem1)
    osems = (osem0, osem1)

    pltpu.sync_copy(ids_hbm.at[:, pl.ds(wid * 512, 512)], ids_v)

    def stage_a(f, p):
        @plsc.parallel_loop(0, 32, unroll=4)
        def _idx(i):
            sl = pl.ds(i * 16, 16)
            v = ids_v[f, sl]
            idx1_v[p, sl] = lax.bitwise_and(v, jnp.int32(65535)) + jnp.int32(_C0)
            h = lax.shift_right_logical(v, 16)
            x = h * jnp.int32(1024) + jnp.where(
                h >= jnp.int32(32768), jnp.int32(511), jnp.int32(0))
            q = (x.astype(jnp.float32) * jnp.float32(1.0 / 15625)).astype(jnp.int32)
            r = x - q * jnp.int32(15625)
            r = jnp.where(r < 0, r + jnp.int32(15625), r)
            r = jnp.where(r >= jnp.int32(15625), r - jnp.int32(15625), r)
            idx0_v[p, sl] = r

        for j in range(4):
            jsl = pl.ds(j * 128, 128)
            pltpu.async_copy(ctab_hbm.at[idx0_v.at[p, jsl]], buf0.at[p, jsl],
                             gsems[p])
            pltpu.async_copy(ctab_hbm.at[idx1_v.at[p, jsl]], buf1.at[p, jsl],
                             gsems[p])

    def stage_b(f, p, first):
        for j in range(4):
            jsl = pl.ds(j * 128, 128)
            pltpu.make_async_copy(ctab_hbm.at[pl.ds(0, 128)], buf0.at[p, jsl],
                                  gsems[p]).wait()
            pltpu.make_async_copy(ctab_hbm.at[pl.ds(0, 128)], buf1.at[p, jsl],
                                  gsems[p]).wait()

        def drain_out():
            for dt in range(4):
                pltpu.make_async_copy(out_hbm.at[0, 0, 0], vout.at[p, dt],
                                      osems[p]).wait()

        if first is None:
            drain_out()
        else:
            pl.when(first)(drain_out)

        @plsc.parallel_loop(0, 512, unroll=4)
        def _asm(r_):
            btl = jnp.full((16,), 0, jnp.int32) + lax.shift_right_logical(r_, 7)
            bi = jnp.full((16,), 0, jnp.int32) + lax.bitwise_and(r_, 127)
            for half in range(2):
                sl = pl.ds(half * 16, 16)
                s = buf0[p, r_, sl] + buf1[p, r_, sl]
                d = half * 16 + lax.iota(jnp.int32, 16)
                plsc.store_scatter(
                    vout.at[p],
                    [lax.shift_right_logical(d, 3), btl,
                     lax.bitwise_and(d, 7), bi],
                    s)

        for dt in range(4):
            pltpu.async_copy(vout.at[p, dt],
                             out_hbm.at[f, dt, pl.ds(wid * _BPW, _BPW)],
                             osems[p])

    stage_a(jnp.int32(0), 0)
    stage_a(jnp.int32(1), 1)

    def outer(go, carry):
        f0 = 2 * go
        for k in range(2):
            stage_b(f0 + k, k, go > 0)
            stage_a(f0 + k + 2, k)
        return carry

    lax.fori_loop(0, (_F - 2) // 2, outer, 0)

    stage_b(jnp.int32(24), 0, None)
    stage_b(jnp.int32(25), 1, None)
    for p in (0, 1):
        for dt in range(4):
            pltpu.make_async_copy(out_hbm.at[0, 0, 0], vout.at[p, dt],
                                  osems[p]).wait()


def kernel(ids, table):
    c0 = table[::64]
    c1 = table[:_C1]
    pad = jnp.zeros((_CC - _C0 - _C1, _D), jnp.float32)
    cc = jnp.concatenate([c0, c1, pad], axis=0)
    lin = _relayout(cc.T)
    out5 = _gather(ids.T, lin.reshape(_CC, _D))
    return out5.transpose(2, 4, 0, 1, 3).reshape(_B, _F, _D)


# speedup vs baseline: 4.8458x; 1.0004x over previous
import functools

import jax
import jax.numpy as jnp
from jax import lax
from jax.experimental import pallas as pl
from jax.experimental.pallas import tpu as pltpu
from jax.experimental.pallas import tpu_sc as plsc

_D = 32
_B = 16384
_F = 26
_NW = 32
_C0 = 15625
_C1 = 65536
_CC = 81280
_VB = _CC // 128
_BT = _B // 128
_UNITS = _F * _BT
_UPW = _UNITS // _NW
_BPW = 4

_mesh = plsc.VectorSubcoreMesh(core_axis_name="c", subcore_axis_name="s")


@functools.partial(
    pl.kernel,
    out_type=jax.ShapeDtypeStruct((_CC * _D // 128, 128), jnp.float32),
    mesh=_mesh,
    compiler_params=pltpu.CompilerParams(use_tc_tiling_on_sc=True, needs_layout_passes=False),
    scratch_types=[
        pltpu.VMEM((2, 4, 8, 128), jnp.float32),
        pltpu.VMEM((2, 32, 128), jnp.float32),
        pltpu.SemaphoreType.DMA,
        pltpu.SemaphoreType.DMA,
        pltpu.SemaphoreType.DMA,
        pltpu.SemaphoreType.DMA,
    ],
)
def _relayout(src_hbm, lin_hbm, vin, vout, gsem0, gsem1, osem0, osem1):
    cid = lax.axis_index("c")
    sid = lax.axis_index("s")
    wid = sid * 2 + cid
    gsems = (gsem0, gsem1)
    osems = (osem0, osem1)
    nfull = 19

    def fire(n):
        p = n % 2
        vb = wid + 32 * n
        for dt in range(4):
            pltpu.async_copy(
                src_hbm.at[pl.ds(dt * 8, 8), pl.ds(vb * 128, 128)],
                vin.at[p, dt], gsems[p])

    def retire(n):
        p = n % 2
        vb = wid + 32 * n
        for dt in range(4):
            pltpu.make_async_copy(lin_hbm.at[pl.ds(0, 8)], vin.at[p, dt],
                                  gsems[p]).wait()
        if n >= 2:
            pltpu.make_async_copy(lin_hbm.at[pl.ds(0, 32)], vout.at[p],
                                  osems[p]).wait()

        @plsc.parallel_loop(0, 256, unroll=4)
        def _asm(j):
            mr = lax.shift_right_logical(j, 3)
            jj = lax.bitwise_and(j, 7)
            lane0 = jj * 16
            d = lax.bitwise_and(lane0, 31) + lax.iota(jnp.int32, 16)
            dt_v = lax.shift_right_logical(d, 3)
            di_v = lax.bitwise_and(d, 7)
            vi = jnp.full((16,), 0, jnp.int32) + (mr * 4 + lax.shift_right_logical(jj, 1))
            vout[p, mr, pl.ds(lane0, 16)] = plsc.load_gather(
                vin.at[p], [dt_v, di_v, vi])

        pltpu.async_copy(vout.at[p], lin_hbm.at[pl.ds(vb * 32, 32)], osems[p])

    fire(0)
    for n in range(1, nfull):
        fire(n)
        retire(n - 1)

    @pl.when(wid < 27)
    def _():
        fire(19)
    retire(nfull - 1)

    @pl.when(wid < 27)
    def _():
        retire(19)

    pltpu.make_async_copy(lin_hbm.at[pl.ds(0, 32)], vout.at[0],
                          osems[0]).wait()
    pltpu.make_async_copy(lin_hbm.at[pl.ds(0, 32)], vout.at[1],
                          osems[1]).wait()


@functools.partial(
    pl.kernel,
    out_type=jax.ShapeDtypeStruct((_F, 4, 128, 1024), jnp.float32),
    mesh=_mesh,
    compiler_params=pltpu.CompilerParams(use_tc_tiling_on_sc=False, needs_layout_passes=False),
    scratch_types=[
        pltpu.VMEM((_F, 512), jnp.int32),
        pltpu.VMEM((2, 512), jnp.int32),
        pltpu.VMEM((2, 512), jnp.int32),
        pltpu.VMEM((2, 512, _D), jnp.float32),
        pltpu.VMEM((2, 512, _D), jnp.float32),
        pltpu.VMEM((2, 4, 4, 1024), jnp.float32),
        pltpu.SemaphoreType.DMA,
        pltpu.SemaphoreType.DMA,
        pltpu.SemaphoreType.DMA,
        pltpu.SemaphoreType.DMA,
    ],
)
def _gather(ids_hbm, ctab_hbm, out_hbm, ids_v, idx0_v, idx1_v, buf0, buf1,
            vout, gsem0, gsem1, osem0, osem1):
    cid = lax.axis_index("c")
    sid = lax.axis_index("s")
    wid = sid * 2 + cid
    gsems = (gsem0, gsem1)
    osems = (osem0, osem1)

    pltpu.sync_copy(ids_hbm.at[:, pl.ds(wid * 512, 512)], ids_v)

    iota = lax.iota(jnp.int32, 16)
    cdt = tuple(lax.shift_right_logical(h * 16 + iota, 3) for h in range(2))
    clo = tuple(lax.bitwise_and(h * 16 + iota, 7) * 128 for h in range(2))

    def stage_a(f, p):
        @plsc.parallel_loop(0, 32, unroll=4)
        def _idx(i):
            sl = pl.ds(i * 16, 16)
            v = ids_v[f, sl]
            idx1_v[p, sl] = lax.bitwise_and(v, jnp.int32(65535)) + jnp.int32(_C0)
            h = lax.shift_right_logical(v, 16)
            x = h * jnp.int32(1024) + jnp.where(
                h >= jnp.int32(32768), jnp.int32(511), jnp.int32(0))
            q = (x.astype(jnp.float32) * jnp.float32(1.0 / 15625)).astype(jnp.int32)
            r = x - q * jnp.int32(15625)
            r = jnp.where(r < 0, r + jnp.int32(15625), r)
            r = jnp.where(r >= jnp.int32(15625), r - jnp.int32(15625), r)
            idx0_v[p, sl] = r

        for j in range(4):
            jsl = pl.ds(j * 128, 128)
            pltpu.async_copy(ctab_hbm.at[idx0_v.at[p, jsl]], buf0.at[p, jsl],
                             gsems[p])
            pltpu.async_copy(ctab_hbm.at[idx1_v.at[p, jsl]], buf1.at[p, jsl],
                             gsems[p])

    def stage_b(f, p, first):
        for j in range(4):
            jsl = pl.ds(j * 128, 128)
            pltpu.make_async_copy(ctab_hbm.at[pl.ds(0, 128)], buf0.at[p, jsl],
                                  gsems[p]).wait()
            pltpu.make_async_copy(ctab_hbm.at[pl.ds(0, 128)], buf1.at[p, jsl],
                                  gsems[p]).wait()

        def drain_out():
            for dt in range(4):
                pltpu.make_async_copy(out_hbm.at[0, 0, pl.ds(0, _BPW)],
                                      vout.at[p, dt], osems[p]).wait()

        if first is None:
            drain_out()
        else:
            pl.when(first)(drain_out)

        @plsc.parallel_loop(0, 512, unroll=4)
        def _asm(r_):
            btl = jnp.full((16,), 0, jnp.int32) + lax.shift_right_logical(r_, 7)
            bi = jnp.full((16,), 0, jnp.int32) + lax.bitwise_and(r_, 127)
            for half in range(2):
                sl = pl.ds(half * 16, 16)
                s = buf0[p, r_, sl] + buf1[p, r_, sl]
                plsc.store_scatter(vout.at[p], [cdt[half], btl, clo[half] + bi], s)

        for dt in range(4):
            pltpu.async_copy(vout.at[p, dt],
                             out_hbm.at[f, dt, pl.ds(wid * _BPW, _BPW)],
                             osems[p])

    stage_a(jnp.int32(0), 0)
    stage_a(jnp.int32(1), 1)

    def outer(go, carry):
        f0 = 2 * go
        for k in range(2):
            stage_b(f0 + k, k, go > 0)
            stage_a(f0 + k + 2, k)
        return carry

    lax.fori_loop(0, (_F - 2) // 2, outer, 0)

    stage_b(jnp.int32(24), 0, None)
    stage_b(jnp.int32(25), 1, None)
    for p in (0, 1):
        for dt in range(4):
            pltpu.make_async_copy(out_hbm.at[0, 0, pl.ds(0, _BPW)],
                                  vout.at[p, dt], osems[p]).wait()


def kernel(ids, table):
    c0 = table[::64]
    c1 = table[:_C1]
    pad = jnp.zeros((_CC - _C0 - _C1, _D), jnp.float32)
    cc = jnp.concatenate([c0, c1, pad], axis=0)
    lin = _relayout(cc.T)
    out4 = _gather(ids.T, lin.reshape(_CC, _D))
    out5 = out4.reshape(_F, 4, 128, 8, 128)
    return out5.transpose(2, 4, 0, 1, 3).reshape(_B, _F, _D)
